# Initial kernel scaffold; baseline (speedup 1.0000x reference)
import jax
import jax.numpy as jnp
from jax.experimental import pallas as pl


def kernel(input_1, input_2, table1, table2, W, b):
    raise NotImplementedError("write your pallas kernel here")


# speedup vs baseline: 12.3162x; 12.3162x over previous
import functools

import jax
import jax.numpy as jnp
from jax import lax
from jax.experimental import pallas as pl
from jax.experimental.pallas import tpu as pltpu
from jax.experimental.pallas import tpu_sc as plsc

B, L = 16384, 200
NV = 10
D = 8
PAIRS = B * (L // 2)
IDX_COLS = 128
IDX_ROWS = PAIRS // IDX_COLS


def _lut_body(t1_ref, t2_ref, w_ref, b_ref, out_ref):
    w = w_ref[...]
    t1 = t1_ref[...]
    t2 = t2_ref[...]
    t1w = sum(t1[:, c:c + 1] * w[c:c + 1, :] for c in range(4))
    t2w = sum(t2[:, c:c + 1] * w[4 + c:5 + c, :] for c in range(4))
    p_row = lax.broadcasted_iota(jnp.int32, (NV * NV, NV), 0)
    p_col = lax.broadcasted_iota(jnp.int32, (NV * NV, NV), 1)
    e_div = jnp.where(p_row // NV == p_col, 1.0, 0.0)
    e_mod = jnp.where(p_row % NV == p_col, 1.0, 0.0)
    lutr = jnp.dot(e_div, t1w, preferred_element_type=jnp.float32)
    lutr = lutr + jnp.dot(e_mod, t2w, preferred_element_type=jnp.float32)
    lutr = jnp.maximum(lutr + b_ref[...], 0.0)
    left = jnp.broadcast_to(lutr[:, None, :], (NV * NV, NV * NV, D))
    right = jnp.broadcast_to(lutr[None, :, :], (NV * NV, NV * NV, D))
    out_ref[...] = jnp.concatenate([left, right], axis=-1)


_lut_call = pl.pallas_call(
    _lut_body,
    out_shape=jax.ShapeDtypeStruct((NV * NV, NV * NV, 2 * D), jnp.float32),
)

_PAIR_BLK = 2048


def _pair_body(i1_ref, i2_ref, out_ref):
    idxc = (i1_ref[...] * NV + i2_ref[...]).astype(jnp.bfloat16)
    l_i = lax.broadcasted_iota(jnp.int32, (L, L // 2), 0)
    p_i = lax.broadcasted_iota(jnp.int32, (L, L // 2), 1)
    s = jnp.where(l_i == 2 * p_i, 100.0,
                  jnp.where(l_i == 2 * p_i + 1, 1.0, 0.0)).astype(jnp.bfloat16)
    out_ref[...] = jnp.dot(idxc, s,
                           preferred_element_type=jnp.float32).astype(jnp.int32)


_pair_call = pl.pallas_call(
    _pair_body,
    grid=(B // _PAIR_BLK,),
    in_specs=[
        pl.BlockSpec((_PAIR_BLK, L), lambda i: (i, 0)),
        pl.BlockSpec((_PAIR_BLK, L), lambda i: (i, 0)),
    ],
    out_specs=pl.BlockSpec((_PAIR_BLK, L // 2), lambda i: (i, 0)),
    out_shape=jax.ShapeDtypeStruct((B, L // 2), jnp.int32),
)

_NC = 2
_NS = 16
_NW = _NC * _NS
_ROWS_PER_W = IDX_ROWS // _NW
_BLK = 16
_NBLK = _ROWS_PER_W // _BLK


@functools.lru_cache(maxsize=None)
def _make_sc_gather():
    mesh = plsc.VectorSubcoreMesh(
        core_axis_name="c", subcore_axis_name="s",
        num_cores=_NC, num_subcores=_NS)

    @functools.partial(
        pl.kernel,
        mesh=mesh,
        out_type=jax.ShapeDtypeStruct((IDX_ROWS, IDX_COLS, 2 * D), jnp.float32),
        scratch_types=[
            pltpu.VMEM((_BLK, IDX_COLS), jnp.int32),
            pltpu.VMEM((_BLK, IDX_COLS, 2 * D), jnp.float32),
            pltpu.SemaphoreType.DMA,
        ],
        compiler_params=pltpu.CompilerParams(use_tc_tiling_on_sc=False),
    )
    def _sc_gather(idx_hbm, lut_hbm, out_hbm, idx_v, rows_v, gsem):
        wid = lax.axis_index("s") * _NC + lax.axis_index("c")
        base = wid * _ROWS_PER_W

        def body(blk, carry):
            row0 = base + blk * _BLK
            pltpu.sync_copy(idx_hbm.at[pl.ds(row0, _BLK)], idx_v)
            copies = [
                pltpu.async_copy(lut_hbm.at[idx_v.at[j]], rows_v.at[j], gsem)
                for j in range(_BLK)
            ]
            for c in copies:
                c.wait()
            pltpu.sync_copy(rows_v, out_hbm.at[pl.ds(row0, _BLK)])
            return carry

        lax.fori_loop(0, _NBLK, body, 0)

    return _sc_gather


def kernel(input_1, input_2, table1, table2, W, b):
    i1 = input_1.astype(jnp.int32)
    i2 = input_2.astype(jnp.int32)
    lut2 = _lut_call(table1, table2, W, b.reshape(1, D))
    idxp = _pair_call(i1, i2)
    out3 = _make_sc_gather()(idxp.reshape(IDX_ROWS, IDX_COLS),
                             lut2.reshape(NV * NV * NV * NV, 2 * D))
    return out3.reshape(B, L, D)


# speedup vs baseline: 28.7782x; 2.3366x over previous
import functools

import jax
import jax.numpy as jnp
from jax import lax
from jax.experimental import pallas as pl
from jax.experimental.pallas import tpu as pltpu
from jax.experimental.pallas import tpu_sc as plsc

B, L = 16384, 200
NV = 10
D = 8
NCODE = NV * NV
LUT_W = 128


def _lutc_body(t1_ref, t2_ref, w_ref, b_ref, out_ref):
    w = w_ref[...]
    t1 = t1_ref[...]
    t2 = t2_ref[...]
    t1w = sum(t1[:, c:c + 1] * w[c:c + 1, :] for c in range(4))
    t2w = sum(t2[:, c:c + 1] * w[4 + c:5 + c, :] for c in range(4))
    p_row = lax.broadcasted_iota(jnp.int32, (NCODE, NV), 0)
    p_col = lax.broadcasted_iota(jnp.int32, (NCODE, NV), 1)
    e_div = jnp.where(p_row // NV == p_col, 1.0, 0.0)
    e_mod = jnp.where(p_row % NV == p_col, 1.0, 0.0)
    lutr = jnp.dot(e_div, t1w, preferred_element_type=jnp.float32)
    lutr = lutr + jnp.dot(e_mod, t2w, preferred_element_type=jnp.float32)
    lutr = jnp.maximum(lutr + b_ref[...], 0.0)
    eye = jnp.where(
        lax.broadcasted_iota(jnp.int32, (NCODE, NCODE), 0)
        == lax.broadcasted_iota(jnp.int32, (NCODE, NCODE), 1), 1.0, 0.0)
    lutc = lax.dot_general(lutr, eye, (((0,), (0,)), ((), ())),
                           preferred_element_type=jnp.float32)
    out_ref[...] = jnp.concatenate(
        [lutc, jnp.zeros((D, LUT_W - NCODE), jnp.float32)], axis=1)


_lutc_call = pl.pallas_call(
    _lutc_body,
    out_shape=jax.ShapeDtypeStruct((D, LUT_W), jnp.float32),
)

_IDX_BLK = 2048


def _idxT_body(i1_ref, i2_ref, out_ref):
    idxc = (i1_ref[...] * NV + i2_ref[...]).astype(jnp.bfloat16)
    eye = jnp.where(
        lax.broadcasted_iota(jnp.int32, (L, L), 0)
        == lax.broadcasted_iota(jnp.int32, (L, L), 1),
        1.0, 0.0).astype(jnp.bfloat16)
    out = lax.dot_general(eye, idxc, (((0,), (1,)), ((), ())),
                          preferred_element_type=jnp.float32)
    out_ref[...] = out.astype(jnp.int32)


_idxT_call = pl.pallas_call(
    _idxT_body,
    grid=(B // _IDX_BLK,),
    in_specs=[
        pl.BlockSpec((_IDX_BLK, L), lambda i: (i, 0)),
        pl.BlockSpec((_IDX_BLK, L), lambda i: (i, 0)),
    ],
    out_specs=pl.BlockSpec((L, _IDX_BLK), lambda i: (0, i)),
    out_shape=jax.ShapeDtypeStruct((L, B), jnp.int32),
)

_NC = 2
_NS = 16
_NW = _NC * _NS
_NROWS = L * D
_ROWS_PER_W = _NROWS // _NW
_NVEC = B // 16


@functools.lru_cache(maxsize=None)
def _make_sc_gather():
    mesh = plsc.VectorSubcoreMesh(
        core_axis_name="c", subcore_axis_name="s",
        num_cores=_NC, num_subcores=_NS)

    @functools.partial(
        pl.kernel,
        mesh=mesh,
        out_type=jax.ShapeDtypeStruct((_NROWS, B), jnp.float32),
        scratch_types=[
            pltpu.VMEM((D, LUT_W), jnp.float32),
            pltpu.VMEM((B,), jnp.int32),
            pltpu.VMEM((B,), jnp.float32),
        ],
        compiler_params=pltpu.CompilerParams(
            use_tc_tiling_on_sc=False, needs_layout_passes=False),
    )
    def _sc_gather(idx_hbm, lutc_hbm, out_hbm, lutc_v, idx_v, out_v):
        wid = lax.axis_index("s") * _NC + lax.axis_index("c")
        base = wid * _ROWS_PER_W
        pltpu.sync_copy(lutc_hbm, lutc_v)

        def row_body(r, carry):
            row = base + r
            l = row // D
            c = row % D
            pltpu.sync_copy(idx_hbm.at[l], idx_v)
            c_vec = jnp.zeros((16,), jnp.int32) + c

            def vec_body(k, carry2):
                vec = idx_v[pl.ds(k * 16, 16)]
                out_v[pl.ds(k * 16, 16)] = plsc.load_gather(
                    lutc_v, [c_vec, vec])
                return carry2

            lax.fori_loop(0, _NVEC, vec_body, 0, unroll=8)
            pltpu.sync_copy(out_v, out_hbm.at[row])
            return carry

        lax.fori_loop(0, _ROWS_PER_W, row_body, 0)

    return _sc_gather


def kernel(input_1, input_2, table1, table2, W, b):
    i1 = input_1.astype(jnp.int32)
    i2 = input_2.astype(jnp.int32)
    lutc = _lutc_call(table1, table2, W, b.reshape(1, D))
    idxT = _idxT_call(i1, i2)
    outT = _make_sc_gather()(idxT, lutc)
    return outT.reshape(L, D, B).transpose(2, 0, 1)


# speedup vs baseline: 53.3685x; 1.8545x over previous
import functools

import jax
import jax.numpy as jnp
from jax import lax
from jax.experimental import pallas as pl
from jax.experimental.pallas import tpu as pltpu
from jax.experimental.pallas import tpu_sc as plsc

B, L = 16384, 200
NV = 10
D = 8
NCODE = NV * NV
LUT_W = 128


def _lutc_body(t1_ref, t2_ref, w_ref, b_ref, out_ref):
    w = w_ref[...]
    t1 = t1_ref[...]
    t2 = t2_ref[...]
    t1w = sum(t1[:, c:c + 1] * w[c:c + 1, :] for c in range(4))
    t2w = sum(t2[:, c:c + 1] * w[4 + c:5 + c, :] for c in range(4))
    p_row = lax.broadcasted_iota(jnp.int32, (NCODE, NV), 0)
    p_col = lax.broadcasted_iota(jnp.int32, (NCODE, NV), 1)
    e_div = jnp.where(p_row // NV == p_col, 1.0, 0.0)
    e_mod = jnp.where(p_row % NV == p_col, 1.0, 0.0)
    lutr = jnp.dot(e_div, t1w, preferred_element_type=jnp.float32)
    lutr = lutr + jnp.dot(e_mod, t2w, preferred_element_type=jnp.float32)
    lutr = jnp.maximum(lutr + b_ref[...], 0.0)
    eye = jnp.where(
        lax.broadcasted_iota(jnp.int32, (NCODE, NCODE), 0)
        == lax.broadcasted_iota(jnp.int32, (NCODE, NCODE), 1), 1.0, 0.0)
    lutc = lax.dot_general(lutr, eye, (((0,), (0,)), ((), ())),
                           preferred_element_type=jnp.float32)
    out_ref[...] = jnp.concatenate(
        [lutc, jnp.zeros((D, LUT_W - NCODE), jnp.float32)], axis=1)


_lutc_call = pl.pallas_call(
    _lutc_body,
    out_shape=jax.ShapeDtypeStruct((D, LUT_W), jnp.float32),
)

_IDX_BLK = 2048


def _idxT_body(i1_ref, i2_ref, out_ref):
    idxc = (i1_ref[...] * NV + i2_ref[...]).astype(jnp.bfloat16)
    eye = jnp.where(
        lax.broadcasted_iota(jnp.int32, (L, L), 0)
        == lax.broadcasted_iota(jnp.int32, (L, L), 1),
        1.0, 0.0).astype(jnp.bfloat16)
    out = lax.dot_general(eye, idxc, (((0,), (1,)), ((), ())),
                          preferred_element_type=jnp.float32)
    out_ref[...] = out.astype(jnp.int32)


_idxT_call = pl.pallas_call(
    _idxT_body,
    grid=(B // _IDX_BLK,),
    in_specs=[
        pl.BlockSpec((_IDX_BLK, L), lambda i: (i, 0)),
        pl.BlockSpec((_IDX_BLK, L), lambda i: (i, 0)),
    ],
    out_specs=pl.BlockSpec((L, _IDX_BLK), lambda i: (0, i)),
    out_shape=jax.ShapeDtypeStruct((L, B), jnp.int32),
)

_NC = 2
_NS = 16
_NW = _NC * _NS
_NROWS = L * D
_CHUNK = 2048
_NCH = B // _CHUNK


@functools.lru_cache(maxsize=None)
def _make_sc_gather():
    mesh = plsc.VectorSubcoreMesh(
        core_axis_name="c", subcore_axis_name="s",
        num_cores=_NC, num_subcores=_NS)

    @functools.partial(
        pl.kernel,
        mesh=mesh,
        out_type=jax.ShapeDtypeStruct((_NROWS, B), jnp.float32),
        scratch_types=[
            pltpu.VMEM((D, LUT_W), jnp.float32),
            pltpu.VMEM((2, B), jnp.int32),
            pltpu.VMEM((2, D, _CHUNK), jnp.float32),
            pltpu.SemaphoreType.DMA,
            pltpu.SemaphoreType.DMA,
        ],
        compiler_params=pltpu.CompilerParams(
            use_tc_tiling_on_sc=False, needs_layout_passes=False),
    )
    def _sc_gather(idx_hbm, lutc_hbm, out_hbm, lutc_v, idx_v2, out_v2,
                   isem, osem):
        wid = lax.axis_index("s") * _NC + lax.axis_index("c")
        l_start = 6 * wid + jnp.minimum(wid, 8)
        n_l = 6 + (wid < 8).astype(jnp.int32)
        pltpu.sync_copy(lutc_hbm, lutc_v)
        c_vecs = [jnp.zeros((16,), jnp.int32) + c for c in range(D)]
        pltpu.make_async_copy(idx_hbm.at[l_start], idx_v2.at[0], isem).start()

        def l_body(li, carry):
            l = l_start + li
            pltpu.make_async_copy(
                idx_hbm.at[l], idx_v2.at[li % 2], isem).wait()

            @pl.when(li + 1 < n_l)
            def _():
                pltpu.make_async_copy(
                    idx_hbm.at[l + 1], idx_v2.at[(li + 1) % 2], isem).start()

            idxbuf = idx_v2.at[li % 2]
            row0 = l * D
            for ch in range(_NCH):
                g = li * _NCH + ch
                obuf = out_v2.at[g % 2]

                @pl.when(g >= 2)
                def _():
                    pltpu.make_async_copy(
                        out_hbm.at[pl.ds(0, D), pl.ds(0, _CHUNK)],
                        out_v2.at[g % 2], osem).wait()

                b0 = ch * _CHUNK

                def k_body(k, cc):
                    vec = idxbuf[pl.ds(b0 + k * 16, 16)]
                    for c in range(D):
                        obuf[c, pl.ds(k * 16, 16)] = plsc.load_gather(
                            lutc_v, [c_vecs[c], vec])
                    return cc

                lax.fori_loop(0, _CHUNK // 16, k_body, 0, unroll=8)
                pltpu.make_async_copy(
                    obuf,
                    out_hbm.at[pl.ds(row0, D), pl.ds(b0, _CHUNK)],
                    osem).start()
            return carry

        lax.fori_loop(0, n_l, l_body, 0)
        for _ in range(2):
            pltpu.make_async_copy(
                out_hbm.at[pl.ds(0, D), pl.ds(0, _CHUNK)],
                out_v2.at[0], osem).wait()

    return _sc_gather


def kernel(input_1, input_2, table1, table2, W, b):
    i1 = input_1.astype(jnp.int32)
    i2 = input_2.astype(jnp.int32)
    lutc = _lutc_call(table1, table2, W, b.reshape(1, D))
    idxT = _idxT_call(i1, i2)
    outT = _make_sc_gather()(idxT, lutc)
    return outT.reshape(L, D, B).transpose(2, 0, 1)


# speedup vs baseline: 53.5796x; 1.0040x over previous
import functools

import jax
import jax.numpy as jnp
from jax import lax
from jax.experimental import pallas as pl
from jax.experimental.pallas import tpu as pltpu
from jax.experimental.pallas import tpu_sc as plsc

B, L = 16384, 200
NV = 10
D = 8
NCODE = NV * NV
LUT_W = 128


def _lutc_body(t1_ref, t2_ref, w_ref, b_ref, out_ref):
    w = w_ref[...]
    t1 = t1_ref[...]
    t2 = t2_ref[...]
    t1w = sum(t1[:, c:c + 1] * w[c:c + 1, :] for c in range(4))
    t2w = sum(t2[:, c:c + 1] * w[4 + c:5 + c, :] for c in range(4))
    p_row = lax.broadcasted_iota(jnp.int32, (NCODE, NV), 0)
    p_col = lax.broadcasted_iota(jnp.int32, (NCODE, NV), 1)
    e_div = jnp.where(p_row // NV == p_col, 1.0, 0.0)
    e_mod = jnp.where(p_row % NV == p_col, 1.0, 0.0)
    lutr = jnp.dot(e_div, t1w, preferred_element_type=jnp.float32)
    lutr = lutr + jnp.dot(e_mod, t2w, preferred_element_type=jnp.float32)
    lutr = jnp.maximum(lutr + b_ref[...], 0.0)
    eye = jnp.where(
        lax.broadcasted_iota(jnp.int32, (NCODE, NCODE), 0)
        == lax.broadcasted_iota(jnp.int32, (NCODE, NCODE), 1), 1.0, 0.0)
    lutc = lax.dot_general(lutr, eye, (((0,), (0,)), ((), ())),
                           preferred_element_type=jnp.float32)
    out_ref[...] = jnp.concatenate(
        [lutc, jnp.zeros((D, LUT_W - NCODE), jnp.float32)], axis=1)


_lutc_call = pl.pallas_call(
    _lutc_body,
    out_shape=jax.ShapeDtypeStruct((D, LUT_W), jnp.float32),
)

_IDX_BLK = 2048


def _idxT_body(i1_ref, i2_ref, out_ref):
    idxc = (i1_ref[...] * NV + i2_ref[...]).astype(jnp.bfloat16)
    eye = jnp.where(
        lax.broadcasted_iota(jnp.int32, (L, L), 0)
        == lax.broadcasted_iota(jnp.int32, (L, L), 1),
        1.0, 0.0).astype(jnp.bfloat16)
    out = lax.dot_general(eye, idxc, (((0,), (1,)), ((), ())),
                          preferred_element_type=jnp.float32)
    out_ref[...] = out.astype(jnp.int32)


_idxT_call = pl.pallas_call(
    _idxT_body,
    grid=(B // _IDX_BLK,),
    in_specs=[
        pl.BlockSpec((_IDX_BLK, L), lambda i: (i, 0)),
        pl.BlockSpec((_IDX_BLK, L), lambda i: (i, 0)),
    ],
    out_specs=pl.BlockSpec((L, _IDX_BLK), lambda i: (0, i)),
    out_shape=jax.ShapeDtypeStruct((L, B), jnp.int32),
)

_NC = 2
_NS = 16
_NW = _NC * _NS
_NROWS = L * D
_CHUNK = 4096
_NCH = B // _CHUNK


@functools.lru_cache(maxsize=None)
def _make_sc_gather():
    mesh = plsc.VectorSubcoreMesh(
        core_axis_name="c", subcore_axis_name="s",
        num_cores=_NC, num_subcores=_NS)

    @functools.partial(
        pl.kernel,
        mesh=mesh,
        out_type=jax.ShapeDtypeStruct((_NROWS, B), jnp.float32),
        scratch_types=[
            pltpu.VMEM((D, LUT_W), jnp.float32),
            pltpu.VMEM((2, B), jnp.int32),
            pltpu.VMEM((2, D, _CHUNK), jnp.float32),
            pltpu.SemaphoreType.DMA,
            pltpu.SemaphoreType.DMA,
        ],
        compiler_params=pltpu.CompilerParams(
            use_tc_tiling_on_sc=False, needs_layout_passes=False),
    )
    def _sc_gather(idx_hbm, lutc_hbm, out_hbm, lutc_v, idx_v2, out_v2,
                   isem, osem):
        wid = lax.axis_index("s") * _NC + lax.axis_index("c")
        l_start = 6 * wid + jnp.minimum(wid, 8)
        n_l = 6 + (wid < 8).astype(jnp.int32)
        pltpu.sync_copy(lutc_hbm, lutc_v)
        c_vecs = [jnp.zeros((16,), jnp.int32) + c for c in range(D)]
        pltpu.make_async_copy(idx_hbm.at[l_start], idx_v2.at[0], isem).start()

        def l_body(li, carry):
            l = l_start + li
            pltpu.make_async_copy(
                idx_hbm.at[l], idx_v2.at[li % 2], isem).wait()

            @pl.when(li + 1 < n_l)
            def _():
                pltpu.make_async_copy(
                    idx_hbm.at[l + 1], idx_v2.at[(li + 1) % 2], isem).start()

            idxbuf = idx_v2.at[li % 2]
            row0 = l * D
            for ch in range(_NCH):
                g = li * _NCH + ch
                obuf = out_v2.at[g % 2]

                @pl.when(g >= 2)
                def _():
                    pltpu.make_async_copy(
                        out_hbm.at[pl.ds(0, D), pl.ds(0, _CHUNK)],
                        out_v2.at[g % 2], osem).wait()

                b0 = ch * _CHUNK

                def k_body(k, cc):
                    vec = idxbuf[pl.ds(b0 + k * 16, 16)]
                    for c in range(D):
                        obuf[c, pl.ds(k * 16, 16)] = plsc.load_gather(
                            lutc_v, [c_vecs[c], vec])
                    return cc

                lax.fori_loop(0, _CHUNK // 16, k_body, 0, unroll=8)
                pltpu.make_async_copy(
                    obuf,
                    out_hbm.at[pl.ds(row0, D), pl.ds(b0, _CHUNK)],
                    osem).start()
            return carry

        lax.fori_loop(0, n_l, l_body, 0)
        for _ in range(2):
            pltpu.make_async_copy(
                out_hbm.at[pl.ds(0, D), pl.ds(0, _CHUNK)],
                out_v2.at[0], osem).wait()

    return _sc_gather


def kernel(input_1, input_2, table1, table2, W, b):
    i1 = input_1.astype(jnp.int32)
    i2 = input_2.astype(jnp.int32)
    lutc = _lutc_call(table1, table2, W, b.reshape(1, D))
    idxT = _idxT_call(i1, i2)
    outT = _make_sc_gather()(idxT, lutc)
    return outT.reshape(L, D, B).transpose(2, 0, 1)


# speedup vs baseline: 102.1776x; 1.9070x over previous
import functools

import jax
import jax.numpy as jnp
from jax import lax
from jax.experimental import pallas as pl
from jax.experimental.pallas import tpu as pltpu
from jax.experimental.pallas import tpu_sc as plsc

B, L = 16384, 200
NV = 10
D = 8
NCODE = NV * NV
LUT_W = 128


def _lutc_body(t1_ref, t2_ref, w_ref, b_ref, out_ref):
    w = w_ref[...]
    t1 = t1_ref[...]
    t2 = t2_ref[...]
    t1w = sum(t1[:, c:c + 1] * w[c:c + 1, :] for c in range(4))
    t2w = sum(t2[:, c:c + 1] * w[4 + c:5 + c, :] for c in range(4))
    p_row = lax.broadcasted_iota(jnp.int32, (NCODE, NV), 0)
    p_col = lax.broadcasted_iota(jnp.int32, (NCODE, NV), 1)
    e_div = jnp.where(p_row // NV == p_col, 1.0, 0.0)
    e_mod = jnp.where(p_row % NV == p_col, 1.0, 0.0)
    lutr = jnp.dot(e_div, t1w, preferred_element_type=jnp.float32)
    lutr = lutr + jnp.dot(e_mod, t2w, preferred_element_type=jnp.float32)
    lutr = jnp.maximum(lutr + b_ref[...], 0.0)
    eye = jnp.where(
        lax.broadcasted_iota(jnp.int32, (NCODE, NCODE), 0)
        == lax.broadcasted_iota(jnp.int32, (NCODE, NCODE), 1), 1.0, 0.0)
    lutc = lax.dot_general(lutr, eye, (((0,), (0,)), ((), ())),
                           preferred_element_type=jnp.float32)
    out_ref[...] = jnp.concatenate(
        [lutc, jnp.zeros((D, LUT_W - NCODE), jnp.float32)], axis=1)


_lutc_call = pl.pallas_call(
    _lutc_body,
    out_shape=jax.ShapeDtypeStruct((D, LUT_W), jnp.float32),
)

_IDX_BLK = 2048


def _idxT_body(i1_ref, i2_ref, out_ref):
    idxc = (i1_ref[...] * NV + i2_ref[...]).astype(jnp.bfloat16)
    eye = jnp.where(
        lax.broadcasted_iota(jnp.int32, (L, L), 0)
        == lax.broadcasted_iota(jnp.int32, (L, L), 1),
        1.0, 0.0).astype(jnp.bfloat16)
    out = lax.dot_general(eye, idxc, (((0,), (1,)), ((), ())),
                          preferred_element_type=jnp.float32)
    out_ref[...] = out.astype(jnp.int32)


_idxT_call = pl.pallas_call(
    _idxT_body,
    grid=(B // _IDX_BLK,),
    in_specs=[
        pl.BlockSpec((_IDX_BLK, L), lambda i: (i, 0)),
        pl.BlockSpec((_IDX_BLK, L), lambda i: (i, 0)),
    ],
    out_specs=pl.BlockSpec((L, _IDX_BLK), lambda i: (0, i)),
    out_shape=jax.ShapeDtypeStruct((L, B), jnp.int32),
)

_NC = 2
_NS = 16
_NW = _NC * _NS
_NROWS = L * D
_CHUNK = 4096
_NCH = B // _CHUNK


@functools.lru_cache(maxsize=None)
def _make_sc_gather():
    mesh = plsc.VectorSubcoreMesh(
        core_axis_name="c", subcore_axis_name="s",
        num_cores=_NC, num_subcores=_NS)

    @functools.partial(
        pl.kernel,
        mesh=mesh,
        out_type=jax.ShapeDtypeStruct((_NROWS, B), jnp.float32),
        scratch_types=[
            pltpu.VMEM((D, LUT_W), jnp.float32),
            pltpu.VMEM((2, B), jnp.int32),
            pltpu.VMEM((2, D, _CHUNK), jnp.float32),
            pltpu.SemaphoreType.DMA,
            pltpu.SemaphoreType.DMA,
        ],
        compiler_params=pltpu.CompilerParams(
            use_tc_tiling_on_sc=False, needs_layout_passes=False),
    )
    def _sc_gather(idx_hbm, lutc_hbm, out_hbm, lutc_v, idx_v2, out_v2,
                   isem, osem):
        wid = lax.axis_index("s") * _NC + lax.axis_index("c")
        l_start = 6 * wid + jnp.minimum(wid, 8)
        n_l = 6 + (wid < 8).astype(jnp.int32)
        pltpu.sync_copy(lutc_hbm, lutc_v)
        c_vecs = [jnp.zeros((16,), jnp.int32) + c for c in range(D)]
        pltpu.make_async_copy(idx_hbm.at[l_start], idx_v2.at[0], isem).start()

        def l_body(li, carry):
            l = l_start + li
            pltpu.make_async_copy(
                idx_hbm.at[l], idx_v2.at[li % 2], isem).wait()

            @pl.when(li + 1 < n_l)
            def _():
                pltpu.make_async_copy(
                    idx_hbm.at[l + 1], idx_v2.at[(li + 1) % 2], isem).start()

            idxbuf = idx_v2.at[li % 2]
            row0 = l * D
            for ch in range(_NCH):
                g = li * _NCH + ch
                obuf = out_v2.at[g % 2]

                @pl.when(g >= 2)
                def _():
                    pltpu.make_async_copy(
                        out_hbm.at[pl.ds(0, D), pl.ds(0, _CHUNK)],
                        out_v2.at[g % 2], osem).wait()

                b0 = ch * _CHUNK

                @plsc.parallel_loop(0, _CHUNK // 16, unroll=8)
                def _(k):
                    vec = idxbuf[pl.ds(b0 + k * 16, 16)]
                    for c in range(D):
                        obuf[c, pl.ds(k * 16, 16)] = plsc.load_gather(
                            lutc_v, [c_vecs[c], vec])
                pltpu.make_async_copy(
                    obuf,
                    out_hbm.at[pl.ds(row0, D), pl.ds(b0, _CHUNK)],
                    osem).start()
            return carry

        lax.fori_loop(0, n_l, l_body, 0)
        for _ in range(2):
            pltpu.make_async_copy(
                out_hbm.at[pl.ds(0, D), pl.ds(0, _CHUNK)],
                out_v2.at[0], osem).wait()

    return _sc_gather


def kernel(input_1, input_2, table1, table2, W, b):
    i1 = input_1.astype(jnp.int32)
    i2 = input_2.astype(jnp.int32)
    lutc = _lutc_call(table1, table2, W, b.reshape(1, D))
    idxT = _idxT_call(i1, i2)
    outT = _make_sc_gather()(idxT, lutc)
    return outT.reshape(L, D, B).transpose(2, 0, 1)


# speedup vs baseline: 186.8807x; 1.8290x over previous
import functools

import jax
import jax.numpy as jnp
from jax import lax
from jax.experimental import pallas as pl
from jax.experimental.pallas import tpu as pltpu
from jax.experimental.pallas import tpu_sc as plsc

B, L = 16384, 200
NV = 10
D = 8
NCODE = NV * NV
LUT_W = 128


def _lutc_body(t1_ref, t2_ref, w_ref, b_ref, out_ref):
    w = w_ref[...]
    t1 = t1_ref[...]
    t2 = t2_ref[...]
    t1w = sum(t1[:, c:c + 1] * w[c:c + 1, :] for c in range(4))
    t2w = sum(t2[:, c:c + 1] * w[4 + c:5 + c, :] for c in range(4))
    p_row = lax.broadcasted_iota(jnp.int32, (NCODE, NV), 0)
    p_col = lax.broadcasted_iota(jnp.int32, (NCODE, NV), 1)
    e_div = jnp.where(p_row // NV == p_col, 1.0, 0.0)
    e_mod = jnp.where(p_row % NV == p_col, 1.0, 0.0)
    lutr = jnp.dot(e_div, t1w, preferred_element_type=jnp.float32)
    lutr = lutr + jnp.dot(e_mod, t2w, preferred_element_type=jnp.float32)
    lutr = jnp.maximum(lutr + b_ref[...], 0.0)
    eye = jnp.where(
        lax.broadcasted_iota(jnp.int32, (NCODE, NCODE), 0)
        == lax.broadcasted_iota(jnp.int32, (NCODE, NCODE), 1), 1.0, 0.0)
    lutc = lax.dot_general(lutr, eye, (((0,), (0,)), ((), ())),
                           preferred_element_type=jnp.float32)
    out_ref[...] = jnp.concatenate(
        [lutc, jnp.zeros((D, LUT_W - NCODE), jnp.float32)], axis=1)


_lutc_call = pl.pallas_call(
    _lutc_body,
    out_shape=jax.ShapeDtypeStruct((D, LUT_W), jnp.float32),
)

_IDX_BLK = 2048


def _idxT_body(i1_ref, i2_ref, out_ref):
    idxc = (i1_ref[...] * NV + i2_ref[...]).astype(jnp.bfloat16)
    eye = jnp.where(
        lax.broadcasted_iota(jnp.int32, (L, L), 0)
        == lax.broadcasted_iota(jnp.int32, (L, L), 1),
        1.0, 0.0).astype(jnp.bfloat16)
    out = lax.dot_general(eye, idxc, (((0,), (1,)), ((), ())),
                          preferred_element_type=jnp.float32)
    out_ref[...] = out.astype(jnp.int32)


_idxT_call = pl.pallas_call(
    _idxT_body,
    grid=(B // _IDX_BLK,),
    in_specs=[
        pl.BlockSpec((_IDX_BLK, L), lambda i: (i, 0)),
        pl.BlockSpec((_IDX_BLK, L), lambda i: (i, 0)),
    ],
    out_specs=pl.BlockSpec((L, _IDX_BLK), lambda i: (0, i)),
    out_shape=jax.ShapeDtypeStruct((L, B), jnp.int32),
)

_NC = 2
_NS = 16
_NW = _NC * _NS
_NROWS = L * D
_CHUNK = 4096
_NCH = B // _CHUNK


@functools.lru_cache(maxsize=None)
def _make_sc_gather():
    mesh = plsc.VectorSubcoreMesh(
        core_axis_name="c", subcore_axis_name="s",
        num_cores=_NC, num_subcores=_NS)

    @functools.partial(
        pl.kernel,
        mesh=mesh,
        out_type=jax.ShapeDtypeStruct((L, B // 128, D, 128), jnp.float32),
        scratch_types=[
            pltpu.VMEM((D, LUT_W), jnp.float32),
            pltpu.VMEM((2, B), jnp.int32),
            pltpu.VMEM((2, _CHUNK // 128, D, 128), jnp.float32),
            pltpu.SemaphoreType.DMA,
            pltpu.SemaphoreType.DMA,
        ],
        compiler_params=pltpu.CompilerParams(
            use_tc_tiling_on_sc=False, needs_layout_passes=False),
    )
    def _sc_gather(idx_hbm, lutc_hbm, out_hbm, lutc_v, idx_v2, out_v2,
                   isem, osem):
        wid = lax.axis_index("s") * _NC + lax.axis_index("c")
        l_start = 6 * wid + jnp.minimum(wid, 8)
        n_l = 6 + (wid < 8).astype(jnp.int32)
        pltpu.sync_copy(lutc_hbm, lutc_v)
        c_vecs = [jnp.zeros((16,), jnp.int32) + c for c in range(D)]
        pltpu.make_async_copy(idx_hbm.at[l_start], idx_v2.at[0], isem).start()

        def l_body(li, carry):
            l = l_start + li
            pltpu.make_async_copy(
                idx_hbm.at[l], idx_v2.at[li % 2], isem).wait()

            @pl.when(li + 1 < n_l)
            def _():
                pltpu.make_async_copy(
                    idx_hbm.at[l + 1], idx_v2.at[(li + 1) % 2], isem).start()

            idxbuf = idx_v2.at[li % 2]
            nt = _CHUNK // 128
            for ch in range(_NCH):
                g = li * _NCH + ch
                obuf = out_v2.at[g % 2]

                @pl.when(g >= 2)
                def _():
                    pltpu.make_async_copy(
                        out_hbm.at[0, pl.ds(0, nt)],
                        out_v2.at[g % 2], osem).wait()

                b0 = ch * _CHUNK

                @plsc.parallel_loop(0, _CHUNK // 16, unroll=8)
                def _(k):
                    vec = idxbuf[pl.ds(b0 + k * 16, 16)]
                    for c in range(D):
                        obuf[k // 8, c, pl.ds((k % 8) * 16, 16)] = (
                            plsc.load_gather(lutc_v, [c_vecs[c], vec]))
                pltpu.make_async_copy(
                    obuf,
                    out_hbm.at[l, pl.ds(ch * nt, nt)],
                    osem).start()
            return carry

        lax.fori_loop(0, n_l, l_body, 0)
        for _ in range(2):
            pltpu.make_async_copy(
                out_hbm.at[0, pl.ds(0, _CHUNK // 128)],
                out_v2.at[0], osem).wait()

    return _sc_gather


def kernel(input_1, input_2, table1, table2, W, b):
    i1 = input_1.astype(jnp.int32)
    i2 = input_2.astype(jnp.int32)
    lutc = _lutc_call(table1, table2, W, b.reshape(1, D))
    idxT = _idxT_call(i1, i2)
    out4 = _make_sc_gather()(idxT, lutc)
    return out4.transpose(1, 3, 0, 2).reshape(B, L, D)


# speedup vs baseline: 202.7866x; 1.0851x over previous
import functools

import jax
import jax.numpy as jnp
from jax import lax
from jax.experimental import pallas as pl
from jax.experimental.pallas import tpu as pltpu
from jax.experimental.pallas import tpu_sc as plsc

B, L = 16384, 200
NV = 10
D = 8
NCODE = NV * NV
LUT_W = 128


def _lutc_body(t1_ref, t2_ref, w_ref, b_ref, out_ref):
    w = w_ref[...]
    t1 = t1_ref[...]
    t2 = t2_ref[...]
    t1w = sum(t1[:, c:c + 1] * w[c:c + 1, :] for c in range(4))
    t2w = sum(t2[:, c:c + 1] * w[4 + c:5 + c, :] for c in range(4))
    p_row = lax.broadcasted_iota(jnp.int32, (NCODE, NV), 0)
    p_col = lax.broadcasted_iota(jnp.int32, (NCODE, NV), 1)
    e_div = jnp.where(p_row // NV == p_col, 1.0, 0.0)
    e_mod = jnp.where(p_row % NV == p_col, 1.0, 0.0)
    lutr = jnp.dot(e_div, t1w, preferred_element_type=jnp.float32)
    lutr = lutr + jnp.dot(e_mod, t2w, preferred_element_type=jnp.float32)
    lutr = jnp.maximum(lutr + b_ref[...], 0.0)
    eye = jnp.where(
        lax.broadcasted_iota(jnp.int32, (NCODE, NCODE), 0)
        == lax.broadcasted_iota(jnp.int32, (NCODE, NCODE), 1), 1.0, 0.0)
    lutc = lax.dot_general(lutr, eye, (((0,), (0,)), ((), ())),
                           preferred_element_type=jnp.float32)
    out_ref[...] = jnp.concatenate(
        [lutc, jnp.zeros((D, LUT_W - NCODE), jnp.float32)], axis=1)


_lutc_call = pl.pallas_call(
    _lutc_body,
    out_shape=jax.ShapeDtypeStruct((D, LUT_W), jnp.float32),
)

_IDX_BLK = 2048


def _idxT_body(i1_ref, i2_ref, out_ref):
    idxc = (i1_ref[...] * NV + i2_ref[...]).astype(jnp.bfloat16)
    eye = jnp.where(
        lax.broadcasted_iota(jnp.int32, (L, L), 0)
        == lax.broadcasted_iota(jnp.int32, (L, L), 1),
        1.0, 0.0).astype(jnp.bfloat16)
    out = lax.dot_general(eye, idxc, (((0,), (1,)), ((), ())),
                          preferred_element_type=jnp.float32)
    out_ref[...] = out.astype(jnp.int32)


_idxT_call = pl.pallas_call(
    _idxT_body,
    grid=(B // _IDX_BLK,),
    in_specs=[
        pl.BlockSpec((_IDX_BLK, L), lambda i: (i, 0)),
        pl.BlockSpec((_IDX_BLK, L), lambda i: (i, 0)),
    ],
    out_specs=pl.BlockSpec((L, _IDX_BLK), lambda i: (0, i)),
    out_shape=jax.ShapeDtypeStruct((L, B), jnp.int32),
)

_NC = 2
_NS = 16
_NW = _NC * _NS
_NROWS = L * D
_CHUNK = 4096
_NCH = B // _CHUNK


@functools.lru_cache(maxsize=None)
def _make_sc_gather():
    mesh = plsc.VectorSubcoreMesh(
        core_axis_name="c", subcore_axis_name="s",
        num_cores=_NC, num_subcores=_NS)

    @functools.partial(
        pl.kernel,
        mesh=mesh,
        out_type=jax.ShapeDtypeStruct((L, B // 128, D, 128), jnp.float32),
        scratch_types=[
            pltpu.VMEM((D, LUT_W), jnp.float32),
            pltpu.VMEM((2, B // 128, 128), jnp.int32),
            pltpu.VMEM((2, _CHUNK // 128, D, 128), jnp.float32),
            pltpu.SemaphoreType.DMA,
            pltpu.SemaphoreType.DMA,
        ],
        compiler_params=pltpu.CompilerParams(
            use_tc_tiling_on_sc=False, needs_layout_passes=False),
    )
    def _sc_gather(idx_hbm, lutc_hbm, out_hbm, lutc_v, idx_v2, out_v2,
                   isem, osem):
        wid = lax.axis_index("s") * _NC + lax.axis_index("c")
        l_start = 6 * wid + jnp.minimum(wid, 8)
        n_l = 6 + (wid < 8).astype(jnp.int32)
        pltpu.sync_copy(lutc_hbm, lutc_v)
        c_vecs = [jnp.zeros((16,), jnp.int32) + c for c in range(D)]

        def idx_copy(l, buf):
            return pltpu.make_async_copy(
                idx_hbm.at[l // D, :, l % D, :], idx_v2.at[buf], isem)

        idx_copy(l_start, 0).start()

        def l_body(li, carry):
            l = l_start + li
            idx_copy(l, li % 2).wait()

            @pl.when(li + 1 < n_l)
            def _():
                idx_copy(l + 1, (li + 1) % 2).start()

            idxbuf = idx_v2.at[li % 2]
            nt = _CHUNK // 128
            for ch in range(_NCH):
                g = li * _NCH + ch
                obuf = out_v2.at[g % 2]

                @pl.when(g >= 2)
                def _():
                    pltpu.make_async_copy(
                        out_hbm.at[0, pl.ds(0, nt)],
                        out_v2.at[g % 2], osem).wait()

                kk0 = ch * (_CHUNK // 16)

                @plsc.parallel_loop(0, _CHUNK // 16, unroll=8)
                def _(k):
                    kk = kk0 + k
                    vec = idxbuf[kk // 8, pl.ds((kk % 8) * 16, 16)]
                    for c in range(D):
                        obuf[k // 8, c, pl.ds((k % 8) * 16, 16)] = (
                            plsc.load_gather(lutc_v, [c_vecs[c], vec]))
                pltpu.make_async_copy(
                    obuf,
                    out_hbm.at[l, pl.ds(ch * nt, nt)],
                    osem).start()
            return carry

        lax.fori_loop(0, n_l, l_body, 0)
        for _ in range(2):
            pltpu.make_async_copy(
                out_hbm.at[0, pl.ds(0, _CHUNK // 128)],
                out_v2.at[0], osem).wait()

    return _sc_gather


def kernel(input_1, input_2, table1, table2, W, b):
    i1 = input_1.astype(jnp.int32)
    i2 = input_2.astype(jnp.int32)
    lutc = _lutc_call(table1, table2, W, b.reshape(1, D))
    idxT = _idxT_call(i1, i2)
    idx4 = idxT.reshape(L // D, D, B // 128, 128).transpose(0, 2, 1, 3)
    out4 = _make_sc_gather()(idx4, lutc)
    return out4.transpose(1, 3, 0, 2).reshape(B, L, D)


# speedup vs baseline: 210.2456x; 1.0368x over previous
import functools

import jax
import jax.numpy as jnp
from jax import lax
from jax.experimental import pallas as pl
from jax.experimental.pallas import tpu as pltpu
from jax.experimental.pallas import tpu_sc as plsc

B, L = 16384, 200
NV = 10
D = 8
NCODE = NV * NV
LUT_W = 128


def _lutc_body(t1_ref, t2_ref, w_ref, b_ref, out_ref):
    w = w_ref[...]
    t1 = t1_ref[...]
    t2 = t2_ref[...]
    t1w = sum(t1[:, c:c + 1] * w[c:c + 1, :] for c in range(4))
    t2w = sum(t2[:, c:c + 1] * w[4 + c:5 + c, :] for c in range(4))
    p_row = lax.broadcasted_iota(jnp.int32, (NCODE, NV), 0)
    p_col = lax.broadcasted_iota(jnp.int32, (NCODE, NV), 1)
    e_div = jnp.where(p_row // NV == p_col, 1.0, 0.0)
    e_mod = jnp.where(p_row % NV == p_col, 1.0, 0.0)
    lutr = jnp.dot(e_div, t1w, preferred_element_type=jnp.float32)
    lutr = lutr + jnp.dot(e_mod, t2w, preferred_element_type=jnp.float32)
    lutr = jnp.maximum(lutr + b_ref[...], 0.0)
    eye = jnp.where(
        lax.broadcasted_iota(jnp.int32, (NCODE, NCODE), 0)
        == lax.broadcasted_iota(jnp.int32, (NCODE, NCODE), 1), 1.0, 0.0)
    lutc = lax.dot_general(lutr, eye, (((0,), (0,)), ((), ())),
                           preferred_element_type=jnp.float32)
    out_ref[...] = jnp.concatenate(
        [lutc, jnp.zeros((D, LUT_W - NCODE), jnp.float32)], axis=1)


_lutc_call = pl.pallas_call(
    _lutc_body,
    out_shape=jax.ShapeDtypeStruct((D, LUT_W), jnp.float32),
)

_IDX_BLK = 2048


def _idxT_body(i1_ref, i2_ref, out_ref):
    idxc = (i1_ref[...] * NV + i2_ref[...]).astype(jnp.bfloat16)
    eye = jnp.where(
        lax.broadcasted_iota(jnp.int32, (L, L), 0)
        == lax.broadcasted_iota(jnp.int32, (L, L), 1),
        1.0, 0.0).astype(jnp.bfloat16)
    out = lax.dot_general(eye, idxc, (((0,), (1,)), ((), ())),
                          preferred_element_type=jnp.float32)
    out_ref[...] = out.astype(jnp.int32)


_idxT_call = pl.pallas_call(
    _idxT_body,
    grid=(B // _IDX_BLK,),
    in_specs=[
        pl.BlockSpec((_IDX_BLK, L), lambda i: (i, 0)),
        pl.BlockSpec((_IDX_BLK, L), lambda i: (i, 0)),
    ],
    out_specs=pl.BlockSpec((L, _IDX_BLK), lambda i: (0, i)),
    out_shape=jax.ShapeDtypeStruct((L, B), jnp.int32),
)

_NC = 2
_NS = 16
_NW = _NC * _NS
_NROWS = L * D
_CHUNK = 4096
_NCH = B // _CHUNK


@functools.lru_cache(maxsize=None)
def _make_sc_gather():
    mesh = plsc.VectorSubcoreMesh(
        core_axis_name="c", subcore_axis_name="s",
        num_cores=_NC, num_subcores=_NS)

    @functools.partial(
        pl.kernel,
        mesh=mesh,
        out_type=jax.ShapeDtypeStruct((L, B // 128, D, 128), jnp.float32),
        scratch_types=[
            pltpu.VMEM((D, LUT_W), jnp.float32),
            pltpu.VMEM((2, B // 256, 128), jnp.int32),
            pltpu.VMEM((2, _CHUNK // 128, D, 128), jnp.float32),
            pltpu.SemaphoreType.DMA,
            pltpu.SemaphoreType.DMA,
        ],
        compiler_params=pltpu.CompilerParams(
            use_tc_tiling_on_sc=False, needs_layout_passes=False),
    )
    def _sc_gather(idx_hbm, lutc_hbm, out_hbm, lutc_v, idx_v2, out_v2,
                   isem, osem):
        wid = lax.axis_index("s") * _NC + lax.axis_index("c")
        u_start = 12 * wid + jnp.minimum(wid, 16)
        n_u = 12 + (wid < 16).astype(jnp.int32)
        nbt = B // 256
        pltpu.sync_copy(lutc_hbm, lutc_v)
        c_vecs = [jnp.zeros((16,), jnp.int32) + c for c in range(D)]

        def idx_copy(u, buf):
            l = u // 2
            h = u % 2
            return pltpu.make_async_copy(
                idx_hbm.at[l // D, pl.ds(h * nbt, nbt), l % D, :],
                idx_v2.at[buf], isem)

        idx_copy(u_start, 0).start()

        def u_body(ui, carry):
            u = u_start + ui
            l = u // 2
            h = u % 2
            idx_copy(u, ui % 2).wait()

            @pl.when(ui + 1 < n_u)
            def _():
                idx_copy(u + 1, (ui + 1) % 2).start()

            idxbuf = idx_v2.at[ui % 2]
            nt = _CHUNK // 128
            nch = B // (2 * _CHUNK)
            for ch in range(nch):
                g = ui * nch + ch
                obuf = out_v2.at[g % 2]

                @pl.when(g >= 2)
                def _():
                    pltpu.make_async_copy(
                        out_hbm.at[0, pl.ds(0, nt)],
                        out_v2.at[g % 2], osem).wait()

                kk0 = ch * (_CHUNK // 16)

                @plsc.parallel_loop(0, _CHUNK // 16, unroll=8)
                def _(k):
                    kk = kk0 + k
                    vec = idxbuf[kk // 8, pl.ds((kk % 8) * 16, 16)]
                    for c in range(D):
                        obuf[k // 8, c, pl.ds((k % 8) * 16, 16)] = (
                            plsc.load_gather(lutc_v, [c_vecs[c], vec]))
                pltpu.make_async_copy(
                    obuf,
                    out_hbm.at[l, pl.ds(h * nbt + ch * nt, nt)],
                    osem).start()
            return carry

        lax.fori_loop(0, n_u, u_body, 0)
        for _ in range(2):
            pltpu.make_async_copy(
                out_hbm.at[0, pl.ds(0, _CHUNK // 128)],
                out_v2.at[0], osem).wait()

    return _sc_gather


def kernel(input_1, input_2, table1, table2, W, b):
    i1 = input_1.astype(jnp.int32)
    i2 = input_2.astype(jnp.int32)
    lutc = _lutc_call(table1, table2, W, b.reshape(1, D))
    idxT = _idxT_call(i1, i2)
    idx4 = idxT.reshape(L // D, D, B // 128, 128).transpose(0, 2, 1, 3)
    out4 = _make_sc_gather()(idx4, lutc)
    return out4.transpose(1, 3, 0, 2).reshape(B, L, D)
